# initial kernel scaffold (unmeasured)
import jax
import jax.numpy as jnp
from jax import lax
from jax.experimental import pallas as pl
from jax.experimental.pallas import tpu as pltpu


def kernel(
    x,
):
    def body(*refs):
        pass

    out_shape = jax.ShapeDtypeStruct(..., jnp.float32)
    return pl.pallas_call(body, out_shape=out_shape)(...)


# baseline (device time: 29142 ns/iter reference)
import jax
import jax.numpy as jnp
from jax import lax
from jax.experimental import pallas as pl
from jax.experimental.pallas import tpu as pltpu

N_Z = 4
M = 512
N_COLS = 2048
CHUNK = N_COLS // N_Z


def kernel(x):
    def body(x_ref, out_ref, send_buf, recv_buf, send_sems, recv_sems):
        my_x = lax.axis_index("x")
        my_y = lax.axis_index("y")
        my_z = lax.axis_index("z")
        right = lax.rem(my_z + 1, N_Z)
        left = lax.rem(my_z + N_Z - 1, N_Z)

        barrier_sem = pltpu.get_barrier_semaphore()
        for nbr in [left, right]:
            pl.semaphore_signal(
                barrier_sem,
                inc=1,
                device_id=(my_x, my_y, nbr),
                device_id_type=pl.DeviceIdType.MESH,
            )
        pl.semaphore_wait(barrier_sem, 2)

        def local_chunk(c):
            return x_ref[0, :, pl.ds(c * CHUNK, CHUNK)]

        send_buf[0, :, :] = local_chunk(lax.rem(my_z + 3, N_Z)).astype(
            jnp.bfloat16
        )
        for h in range(N_Z - 1):
            rdma = pltpu.make_async_remote_copy(
                src_ref=send_buf.at[h],
                dst_ref=recv_buf.at[h],
                send_sem=send_sems.at[h],
                recv_sem=recv_sems.at[h],
                device_id=(my_x, my_y, right),
                device_id_type=pl.DeviceIdType.MESH,
            )
            rdma.start()
            rdma.wait()
            c = lax.rem(my_z + 2 - h + N_Z, N_Z)
            if h < N_Z - 2:
                send_buf[h + 1, :, :] = recv_buf[h] + local_chunk(c).astype(
                    jnp.bfloat16
                )
            else:
                out_ref[:, :] = recv_buf[h].astype(jnp.float32) + local_chunk(c)

    return pl.pallas_call(
        body,
        out_shape=jax.ShapeDtypeStruct((M, CHUNK), jnp.float32),
        in_specs=[pl.BlockSpec(memory_space=pltpu.VMEM)],
        out_specs=pl.BlockSpec(memory_space=pltpu.VMEM),
        scratch_shapes=[
            pltpu.VMEM((N_Z - 1, M, CHUNK), jnp.bfloat16),
            pltpu.VMEM((N_Z - 1, M, CHUNK), jnp.bfloat16),
            pltpu.SemaphoreType.DMA((N_Z - 1,)),
            pltpu.SemaphoreType.DMA((N_Z - 1,)),
        ],
        compiler_params=pltpu.CompilerParams(collective_id=0),
    )(x)


# device time: 19224 ns/iter; 1.5159x vs baseline; 1.5159x over previous
import jax
import jax.numpy as jnp
from jax import lax
from jax.experimental import pallas as pl
from jax.experimental.pallas import tpu as pltpu

N_Z = 4
M = 512
N_COLS = 2048
CHUNK = N_COLS // N_Z
QW = CHUNK // 4


def kernel(x):
    def body(
        x_ref,
        out_ref,
        send_z,
        recv_z,
        send_xy,
        recv_xy,
        send_z_sems,
        recv_z_sems,
        send_xy_sems,
        recv_xy_sems,
    ):
        my_x = lax.axis_index("x")
        my_y = lax.axis_index("y")
        my_z = lax.axis_index("z")
        q = 2 * my_x + my_y

        z_peers = [(my_x, my_y, lax.rem(my_z + dz, N_Z)) for dz in (1, 2, 3)]
        xy_peers = [
            (1 - my_x, my_y, my_z),
            (my_x, 1 - my_y, my_z),
            (1 - my_x, 1 - my_y, my_z),
        ]

        barrier_sem = pltpu.get_barrier_semaphore()
        for pid in z_peers + xy_peers:
            pl.semaphore_signal(
                barrier_sem,
                inc=1,
                device_id=pid,
                device_id_type=pl.DeviceIdType.MESH,
            )
        pl.semaphore_wait(barrier_sem, 6)

        def quarter(c):
            return x_ref[0, :, pl.ds(c * CHUNK + q * QW, QW)]

        z_rdmas = []
        for j, dz in enumerate((1, 2, 3)):
            zt = lax.rem(my_z + dz, N_Z)
            send_z[j, :, :] = quarter(zt).astype(jnp.bfloat16)
            r = pltpu.make_async_remote_copy(
                src_ref=send_z.at[j],
                dst_ref=recv_z.at[2 - j],
                send_sem=send_z_sems.at[j],
                recv_sem=recv_z_sems.at[2 - j],
                device_id=(my_x, my_y, zt),
                device_id_type=pl.DeviceIdType.MESH,
            )
            r.start()
            z_rdmas.append(r)
        for r in z_rdmas:
            r.wait_recv()

        acc = (
            recv_z[0].astype(jnp.float32)
            + recv_z[1].astype(jnp.float32)
            + recv_z[2].astype(jnp.float32)
            + quarter(my_z)
        )
        out_ref[:, pl.ds(q * QW, QW)] = acc
        send_xy[:, :] = acc.astype(jnp.bfloat16)

        xy_rdmas = []
        for s, pid in enumerate(xy_peers):
            r = pltpu.make_async_remote_copy(
                src_ref=send_xy,
                dst_ref=recv_xy.at[s],
                send_sem=send_xy_sems.at[s],
                recv_sem=recv_xy_sems.at[s],
                device_id=pid,
                device_id_type=pl.DeviceIdType.MESH,
            )
            r.start()
            xy_rdmas.append(r)

        partner_q = [
            2 * (1 - my_x) + my_y,
            2 * my_x + (1 - my_y),
            2 * (1 - my_x) + (1 - my_y),
        ]
        for s in range(3):
            xy_rdmas[s].wait_recv()
            out_ref[:, pl.ds(partner_q[s] * QW, QW)] = recv_xy[s].astype(
                jnp.float32
            )

        for r in z_rdmas + xy_rdmas:
            r.wait_send()

    return pl.pallas_call(
        body,
        out_shape=jax.ShapeDtypeStruct((M, CHUNK), jnp.float32),
        in_specs=[pl.BlockSpec(memory_space=pltpu.VMEM)],
        out_specs=pl.BlockSpec(memory_space=pltpu.VMEM),
        scratch_shapes=[
            pltpu.VMEM((3, M, QW), jnp.bfloat16),
            pltpu.VMEM((3, M, QW), jnp.bfloat16),
            pltpu.VMEM((M, QW), jnp.bfloat16),
            pltpu.VMEM((3, M, QW), jnp.bfloat16),
            pltpu.SemaphoreType.DMA((3,)),
            pltpu.SemaphoreType.DMA((3,)),
            pltpu.SemaphoreType.DMA((3,)),
            pltpu.SemaphoreType.DMA((3,)),
        ],
        compiler_params=pltpu.CompilerParams(collective_id=0),
    )(x)


# device time: 18584 ns/iter; 1.5681x vs baseline; 1.0344x over previous
import jax
import jax.numpy as jnp
from jax import lax
from jax.experimental import pallas as pl
from jax.experimental.pallas import tpu as pltpu

N_Z = 4
M = 512
N_COLS = 2048
CHUNK = N_COLS // N_Z
QW = CHUNK // 4
N_SEG = 2
SM = M // N_SEG


def kernel(x):
    def body(
        x_ref,
        out_ref,
        send_z,
        recv_z,
        send_xy,
        recv_xy,
        send_z_sems,
        recv_z_sems,
        send_xy_sems,
        recv_xy_sems,
    ):
        my_x = lax.axis_index("x")
        my_y = lax.axis_index("y")
        my_z = lax.axis_index("z")
        q = 2 * my_x + my_y

        z_peers = [(my_x, my_y, lax.rem(my_z + dz, N_Z)) for dz in (1, 2, 3)]
        xy_peers = [
            (1 - my_x, my_y, my_z),
            (my_x, 1 - my_y, my_z),
            (1 - my_x, 1 - my_y, my_z),
        ]

        barrier_sem = pltpu.get_barrier_semaphore()
        for pid in z_peers + xy_peers:
            pl.semaphore_signal(
                barrier_sem,
                inc=1,
                device_id=pid,
                device_id_type=pl.DeviceIdType.MESH,
            )
        pl.semaphore_wait(barrier_sem, 6)

        def quarter(c, seg):
            return x_ref[
                0,
                seg * SM : (seg + 1) * SM,
                pl.ds(c * CHUNK + q * QW, QW),
            ]

        rows = [slice(seg * SM, (seg + 1) * SM) for seg in range(N_SEG)]

        z_rdmas = [[None] * 3 for _ in range(N_SEG)]
        for seg in range(N_SEG):
            for j, dz in enumerate((1, 2, 3)):
                zt = lax.rem(my_z + dz, N_Z)
                send_z[j, rows[seg], :] = quarter(zt, seg).astype(jnp.bfloat16)
                r = pltpu.make_async_remote_copy(
                    src_ref=send_z.at[j, rows[seg]],
                    dst_ref=recv_z.at[2 - j, rows[seg]],
                    send_sem=send_z_sems.at[seg, j],
                    recv_sem=recv_z_sems.at[seg, 2 - j],
                    device_id=(my_x, my_y, zt),
                    device_id_type=pl.DeviceIdType.MESH,
                )
                r.start()
                z_rdmas[seg][j] = r

        xy_rdmas = [[None] * 3 for _ in range(N_SEG)]
        for seg in range(N_SEG):
            for r in z_rdmas[seg]:
                r.wait_recv()
            acc = (
                recv_z[0, rows[seg], :].astype(jnp.float32)
                + recv_z[1, rows[seg], :].astype(jnp.float32)
                + recv_z[2, rows[seg], :].astype(jnp.float32)
                + quarter(my_z, seg)
            )
            out_ref[rows[seg], pl.ds(q * QW, QW)] = acc
            send_xy[rows[seg], :] = acc.astype(jnp.bfloat16)
            for s, pid in enumerate(xy_peers):
                r = pltpu.make_async_remote_copy(
                    src_ref=send_xy.at[rows[seg]],
                    dst_ref=recv_xy.at[s, rows[seg]],
                    send_sem=send_xy_sems.at[seg, s],
                    recv_sem=recv_xy_sems.at[seg, s],
                    device_id=pid,
                    device_id_type=pl.DeviceIdType.MESH,
                )
                r.start()
                xy_rdmas[seg][s] = r

        partner_q = [
            2 * (1 - my_x) + my_y,
            2 * my_x + (1 - my_y),
            2 * (1 - my_x) + (1 - my_y),
        ]
        for seg in range(N_SEG):
            for s in range(3):
                xy_rdmas[seg][s].wait_recv()
                out_ref[rows[seg], pl.ds(partner_q[s] * QW, QW)] = recv_xy[
                    s, rows[seg], :
                ].astype(jnp.float32)

        for seg in range(N_SEG):
            for r in z_rdmas[seg] + xy_rdmas[seg]:
                r.wait_send()

    return pl.pallas_call(
        body,
        out_shape=jax.ShapeDtypeStruct((M, CHUNK), jnp.float32),
        in_specs=[pl.BlockSpec(memory_space=pltpu.VMEM)],
        out_specs=pl.BlockSpec(memory_space=pltpu.VMEM),
        scratch_shapes=[
            pltpu.VMEM((3, M, QW), jnp.bfloat16),
            pltpu.VMEM((3, M, QW), jnp.bfloat16),
            pltpu.VMEM((M, QW), jnp.bfloat16),
            pltpu.VMEM((3, M, QW), jnp.bfloat16),
            pltpu.SemaphoreType.DMA((N_SEG, 3)),
            pltpu.SemaphoreType.DMA((N_SEG, 3)),
            pltpu.SemaphoreType.DMA((N_SEG, 3)),
            pltpu.SemaphoreType.DMA((N_SEG, 3)),
        ],
        compiler_params=pltpu.CompilerParams(collective_id=0),
    )(x)


# device time: 18398 ns/iter; 1.5840x vs baseline; 1.0101x over previous
import jax
import jax.numpy as jnp
from jax import lax
from jax.experimental import pallas as pl
from jax.experimental.pallas import tpu as pltpu

N_Z = 4
M = 512
N_COLS = 2048
CHUNK = N_COLS // N_Z
QW = CHUNK // 4
N_SEG = 4
SM = M // N_SEG


def kernel(x):
    def body(
        x_ref,
        out_ref,
        send_z,
        recv_z,
        send_xy,
        recv_xy,
        send_z_sems,
        recv_z_sems,
        send_xy_sems,
        recv_xy_sems,
    ):
        my_x = lax.axis_index("x")
        my_y = lax.axis_index("y")
        my_z = lax.axis_index("z")
        q = 2 * my_x + my_y

        z_peers = [(my_x, my_y, lax.rem(my_z + dz, N_Z)) for dz in (1, 2, 3)]
        xy_peers = [
            (1 - my_x, my_y, my_z),
            (my_x, 1 - my_y, my_z),
            (1 - my_x, 1 - my_y, my_z),
        ]

        barrier_sem = pltpu.get_barrier_semaphore()
        for pid in z_peers + xy_peers:
            pl.semaphore_signal(
                barrier_sem,
                inc=1,
                device_id=pid,
                device_id_type=pl.DeviceIdType.MESH,
            )

        def quarter(c, seg):
            return x_ref[
                0,
                seg * SM : (seg + 1) * SM,
                pl.ds(c * CHUNK + q * QW, QW),
            ]

        rows = [slice(seg * SM, (seg + 1) * SM) for seg in range(N_SEG)]

        for j, dz in enumerate((1, 2, 3)):
            zt = lax.rem(my_z + dz, N_Z)
            send_z[j, :, :] = x_ref[
                0, :, pl.ds(zt * CHUNK + q * QW, QW)
            ].astype(jnp.bfloat16)

        pl.semaphore_wait(barrier_sem, 6)

        z_rdmas = [[None] * 3 for _ in range(N_SEG)]
        for seg in range(N_SEG):
            for j, dz in enumerate((1, 2, 3)):
                zt = lax.rem(my_z + dz, N_Z)
                r = pltpu.make_async_remote_copy(
                    src_ref=send_z.at[j, rows[seg]],
                    dst_ref=recv_z.at[2 - j, rows[seg]],
                    send_sem=send_z_sems.at[seg, j],
                    recv_sem=recv_z_sems.at[seg, 2 - j],
                    device_id=(my_x, my_y, zt),
                    device_id_type=pl.DeviceIdType.MESH,
                )
                r.start()
                z_rdmas[seg][j] = r

        xy_rdmas = [[None] * 3 for _ in range(N_SEG)]
        for seg in range(N_SEG):
            for r in z_rdmas[seg]:
                r.wait_recv()
            acc = (
                recv_z[0, rows[seg], :].astype(jnp.float32)
                + recv_z[1, rows[seg], :].astype(jnp.float32)
                + recv_z[2, rows[seg], :].astype(jnp.float32)
                + quarter(my_z, seg)
            )
            send_xy[rows[seg], :] = acc.astype(jnp.bfloat16)
            for s, pid in enumerate(xy_peers):
                r = pltpu.make_async_remote_copy(
                    src_ref=send_xy.at[rows[seg]],
                    dst_ref=recv_xy.at[s, rows[seg]],
                    send_sem=send_xy_sems.at[seg, s],
                    recv_sem=recv_xy_sems.at[seg, s],
                    device_id=pid,
                    device_id_type=pl.DeviceIdType.MESH,
                )
                r.start()
                xy_rdmas[seg][s] = r
            out_ref[rows[seg], pl.ds(q * QW, QW)] = acc

        partner_q = [
            2 * (1 - my_x) + my_y,
            2 * my_x + (1 - my_y),
            2 * (1 - my_x) + (1 - my_y),
        ]
        for seg in range(N_SEG):
            for s in range(3):
                xy_rdmas[seg][s].wait_recv()
                out_ref[rows[seg], pl.ds(partner_q[s] * QW, QW)] = recv_xy[
                    s, rows[seg], :
                ].astype(jnp.float32)

        for seg in range(N_SEG):
            for r in z_rdmas[seg] + xy_rdmas[seg]:
                r.wait_send()

    return pl.pallas_call(
        body,
        out_shape=jax.ShapeDtypeStruct((M, CHUNK), jnp.float32),
        in_specs=[pl.BlockSpec(memory_space=pltpu.VMEM)],
        out_specs=pl.BlockSpec(memory_space=pltpu.VMEM),
        scratch_shapes=[
            pltpu.VMEM((3, M, QW), jnp.bfloat16),
            pltpu.VMEM((3, M, QW), jnp.bfloat16),
            pltpu.VMEM((M, QW), jnp.bfloat16),
            pltpu.VMEM((3, M, QW), jnp.bfloat16),
            pltpu.SemaphoreType.DMA((N_SEG, 3)),
            pltpu.SemaphoreType.DMA((N_SEG, 3)),
            pltpu.SemaphoreType.DMA((N_SEG, 3)),
            pltpu.SemaphoreType.DMA((N_SEG, 3)),
        ],
        compiler_params=pltpu.CompilerParams(collective_id=0),
    )(x)
